# baseline (speedup 1.0000x reference)
import jax
import jax.numpy as jnp
from jax.experimental import pallas as pl
from jax.experimental.pallas import tpu as pltpu

_N_UIDS = 8192
_TOPK = 20
_BATCH = 32
_ROWS = 32 * 128
_D = 512
_QD = 2048
_UID_BLK = 1024
_ROW_BLK = 256
_NG = _N_UIDS // _UID_BLK
_NR = _ROWS // _ROW_BLK
_NS = 4
_PART = _UID_BLK // _NS

_NEG = float("-inf")
_BIGI = 2 ** 30


def _topk_from(mw, tw_ref, ow_ref, rs_ref, wsm_ref):
    ridx = jax.lax.broadcasted_iota(jnp.int32, mw.shape, 0)
    cidx = jax.lax.broadcasted_iota(jnp.int32, mw.shape, 1)
    flat = ridx * 128 + cidx
    vals = mw
    tvals, tidxs = [], []
    for _ in range(_TOPK):
        m = jnp.max(vals)
        i = jnp.min(jnp.where(vals == m, flat, _BIGI))
        tvals.append(m)
        tidxs.append(i)
        vals = jnp.where(flat == i, _NEG, vals)

    lane = jax.lax.broadcasted_iota(jnp.int32, (1, 128), 1)
    tw = jnp.zeros((1, 128), jnp.float32)
    for r in range(_TOPK):
        tw = jnp.where(lane == r, tvals[r], tw)
        wsm_ref[r] = tvals[r]
    tw_ref[...] = tw

    mn = tvals[-1]
    total = tvals[0] - mn
    for r in range(1, _TOPK):
        total = total + (tvals[r] - mn)
    ow = jnp.zeros(mw.shape, jnp.float32)
    member = jnp.zeros(mw.shape, jnp.bool_)
    for r in range(_TOPK):
        hit = flat == tidxs[r]
        ow = jnp.where(hit, (tvals[r] - mn) / total, ow)
        member = jnp.logical_or(member, hit)
    ow_ref[...] = ow
    rs_ref[...] = jnp.where(member, jnp.float32(float(_BATCH)),
                            jnp.float32(0.0))


def _fused_body(qt_ref, w0, w1, w2, w3, b_ref, r_ref,
                o_ref, tw_ref, ow_ref, rs_ref, mw_scr, wsm_ref):
    i = pl.program_id(0)

    @pl.when(i < _NG)
    def _gate():
        dn = (((1,), (0,)), ((), ()))
        for p, w_ref in enumerate((w0, w1, w2, w3)):
            s = jax.lax.dot_general(w_ref[0], qt_ref[...], dn,
                                    preferred_element_type=jnp.float32)
            mv = jnp.sum(s, axis=1) * (1.0 / _BATCH)
            lo = p * _PART
            row = (mv + b_ref[0, 0, lo:lo + _PART]).reshape(_PART // 128, 128)
            mw_scr[pl.ds((i * _UID_BLK + lo) // 128, _PART // 128), :] = row

    @pl.when(i == _NG - 1)
    def _topk():
        _topk_from(mw_scr[...], tw_ref, ow_ref, rs_ref, wsm_ref)

    @pl.when(i >= _NG)
    def _join():
        acc = r_ref[0] * wsm_ref[0]
        for k in range(1, _TOPK):
            acc = acc + r_ref[k] * wsm_ref[k]
        o_ref[...] = acc


def kernel(query, responses, gate_W, gate_b):
    responses3 = responses.reshape(_TOPK, _ROWS, _D)
    qt = jnp.swapaxes(query, 0, 1)
    w4 = gate_W.reshape(_N_UIDS // _PART, _PART, _QD)
    b3 = gate_b.reshape(_NG, 1, _UID_BLK)

    gmax = _NG - 1
    w_specs = [
        pl.BlockSpec(
            (1, _PART, _QD),
            (lambda i, p=p: (_NS * jnp.minimum(i, gmax) + p, 0, 0)))
        for p in range(_NS)
    ]
    weighted, tw, ow, rs = pl.pallas_call(
        _fused_body,
        grid=(_NG + _NR,),
        in_specs=[pl.BlockSpec(qt.shape, lambda i: (0, 0))] + w_specs + [
            pl.BlockSpec((1, 1, _UID_BLK),
                         lambda i: (jnp.minimum(i, gmax), 0, 0)),
            pl.BlockSpec((_TOPK, _ROW_BLK, _D),
                         lambda i: (0, jnp.maximum(i - _NG, 0), 0)),
        ],
        out_specs=[
            pl.BlockSpec((_ROW_BLK, _D), lambda i: (jnp.maximum(i - _NG, 0), 0)),
            pl.BlockSpec((1, 128), lambda i: (0, 0)),
            pl.BlockSpec((_N_UIDS // 128, 128), lambda i: (0, 0)),
            pl.BlockSpec((_N_UIDS // 128, 128), lambda i: (0, 0)),
        ],
        out_shape=[
            jax.ShapeDtypeStruct((_ROWS, _D), jnp.float32),
            jax.ShapeDtypeStruct((1, 128), jnp.float32),
            jax.ShapeDtypeStruct((_N_UIDS // 128, 128), jnp.float32),
            jax.ShapeDtypeStruct((_N_UIDS // 128, 128), jnp.float32),
        ],
        scratch_shapes=[
            pltpu.VMEM((_N_UIDS // 128, 128), jnp.float32),
            pltpu.SMEM((_TOPK,), jnp.float32),
        ],
    )(qt, w4, w4, w4, w4, b3, responses3)
    del tw
    return (weighted.reshape(_BATCH, _ROWS // _BATCH, _D),
            ow.reshape(_N_UIDS), rs.reshape(_N_UIDS))
